# baseline (speedup 1.0000x reference)
import functools

import jax
import jax.numpy as jnp
from jax import lax
from jax.experimental import pallas as pl
from jax.experimental.pallas import tpu as pltpu
from jax.experimental.pallas import tpu_sc as plsc

N = 10000
E = 320000
D = 128
H = 64
L = 16

SUB = 128
NSUBP = 2560
EPAD = NSUBP * SUB
NB = NSUBP // 32
NPAD = 10240
ROWS_PER_TILE = NPAD // 16
HK = H // 16

_mesh = plsc.VectorSubcoreMesh(core_axis_name="c", subcore_axis_name="s")
_sc_params = pltpu.CompilerParams(use_tc_tiling_on_sc=False,
                                  needs_layout_passes=False)


def _seg_sum_body(hw_hbm, sdw_hbm, out_hbm, ebuf, dbuf, rows, acc,
                  esem, gsem, ssem):
    c = lax.axis_index("c")
    s = lax.axis_index("s")
    wid = s * 2 + c

    z16 = jnp.zeros((16,), jnp.float32)

    def zero_body(i, _):
        for k in range(HK):
            rows[0, i, pl.ds(k * 16, 16)] = z16
        return 0

    lax.fori_loop(0, SUB, zero_body, 0)
    for j in range(ROWS_PER_TILE // SUB):
        pltpu.sync_copy(rows.at[0],
                        acc.at[pl.ds(s * ROWS_PER_TILE + j * SUB, SUB)])
    plsc.subcore_barrier()

    def q_of(t):
        return wid + 32 * t

    def issue_edata(t, u):
        pltpu.async_copy(sdw_hbm.at[q_of(t)], ebuf.at[u], esem.at[u])

    def wait_edata(t, u):
        pltpu.make_async_copy(sdw_hbm.at[q_of(t)], ebuf.at[u],
                              esem.at[u]).wait()

    def issue_gather(u):
        pltpu.async_copy(hw_hbm.at[ebuf.at[u, 0]], rows.at[u], gsem.at[u])

    def wait_gather(u):
        pltpu.make_async_copy(hw_hbm.at[ebuf.at[u, 0]], rows.at[u],
                              gsem.at[u]).wait()

    def issue_scatter(u):
        pltpu.async_copy(rows.at[u], acc.at[dbuf.at[u]], ssem.at[u], add=True)

    def wait_scatter(u):
        pltpu.make_async_copy(rows.at[u], acc.at[dbuf.at[u]],
                              ssem.at[u]).wait()

    def scale(u):
        for g in range(SUB // 16):
            w16 = plsc.bitcast(ebuf[u, 2, pl.ds(g * 16, 16)], jnp.float32)
            for i in range(16):
                wb = w16.at[jnp.full((16,), i, jnp.int32)].get(
                    mode="promise_in_bounds")
                e = g * 16 + i
                for k in range(HK):
                    sl = pl.ds(k * 16, 16)
                    rows[u, e, sl] = rows[u, e, sl] * wb

    for u in range(4):
        issue_edata(u, u)
    wait_edata(0, 0)
    issue_gather(0)
    wait_edata(1, 1)
    issue_gather(1)

    def step(t4, u):
        t = t4 * 4 + u
        wait_gather(u)
        for g in range(SUB // 16):
            sl = pl.ds(g * 16, 16)
            dbuf[u, sl] = ebuf[u, 1, sl]
        scale(u)
        issue_scatter(u)

        @pl.when(t4 < (NB // 4) - 1)
        def _():
            issue_edata(t + 4, u)

        un2 = (u + 2) % 4

        def tail_ops(with_scatter_wait):
            if with_scatter_wait:
                wait_scatter(un2)
            wait_edata(t + 2, un2)
            issue_gather(un2)

        if u >= 2:
            @pl.when(t4 < (NB // 4) - 1)
            def _():
                tail_ops(True)
        else:
            @pl.when(t4 >= 1)
            def _():
                tail_ops(True)

            @pl.when(t4 == 0)
            def _():
                tail_ops(False)

    def loop_body(t4, _):
        for u in range(4):
            step(t4, u)
        return 0

    lax.fori_loop(0, NB // 4, loop_body, 0)

    for u in range(4):
        wait_scatter(u)
    plsc.subcore_barrier()
    pltpu.sync_copy(acc.at[pl.ds(s * ROWS_PER_TILE, ROWS_PER_TILE)],
                    out_hbm.at[c, pl.ds(s * ROWS_PER_TILE, ROWS_PER_TILE)])


_seg_sum = pl.kernel(
    _seg_sum_body,
    out_type=jax.ShapeDtypeStruct((2, NPAD, H), jnp.float32),
    mesh=_mesh,
    compiler_params=_sc_params,
    scratch_types=[
        pltpu.VMEM((4, 3, SUB), jnp.int32),
        pltpu.VMEM((4, SUB), jnp.int32),
        pltpu.VMEM((4, SUB, H), jnp.float32),
        pltpu.VMEM_SHARED((NPAD, H), jnp.float32),
        pltpu.SemaphoreType.DMA((4,)),
        pltpu.SemaphoreType.DMA((4,)),
        pltpu.SemaphoreType.DMA((4,)),
    ],
)


def _zdiff_body(z_hbm, sdw_hbm, zd_hbm, ebuf, zib, zjb,
                esem, gisem, gjsem, csem):
    c = lax.axis_index("c")
    s = lax.axis_index("s")
    wid = s * 2 + c

    def q_of(t):
        return wid + 32 * t

    def issue_edata(t, u):
        pltpu.async_copy(sdw_hbm.at[q_of(t)], ebuf.at[u], esem.at[u])

    def wait_edata(t, u):
        pltpu.make_async_copy(sdw_hbm.at[q_of(t)], ebuf.at[u],
                              esem.at[u]).wait()

    def gathers(u):
        pltpu.async_copy(z_hbm.at[ebuf.at[u, 0]], zib.at[u], gisem.at[u])
        pltpu.async_copy(z_hbm.at[ebuf.at[u, 1]], zjb.at[u], gjsem.at[u])

    def wait_gathers(u):
        pltpu.make_async_copy(z_hbm.at[ebuf.at[u, 0]], zib.at[u],
                              gisem.at[u]).wait()
        pltpu.make_async_copy(z_hbm.at[ebuf.at[u, 1]], zjb.at[u],
                              gjsem.at[u]).wait()

    def issue_copyout(t, u):
        pltpu.async_copy(zib.at[u], zd_hbm.at[pl.ds(q_of(t) * SUB, SUB)],
                         csem.at[u])

    def wait_copyout(t, u):
        pltpu.make_async_copy(zib.at[u], zd_hbm.at[pl.ds(q_of(t) * SUB, SUB)],
                              csem.at[u]).wait()

    issue_edata(0, 0)
    issue_edata(1, 1)

    def step(t, u, first):
        wait_edata(t, u)
        gathers(u)
        if first:
            @pl.when(t >= 1)
            def _():
                wait_copyout(t - 1, (u + 1) % 2)
        else:
            wait_copyout(t - 1, (u + 1) % 2)
        wait_gathers(u)
        for e in range(SUB):
            zib[u, e, :] = zib[u, e, :] - zjb[u, e, :]
        issue_copyout(t, u)

        @pl.when(t + 2 < NB)
        def _():
            issue_edata(t + 2, u)

    def loop2(t2, _):
        step(t2 * 2, 0, True)
        step(t2 * 2 + 1, 1, False)
        return 0

    lax.fori_loop(0, NB // 2, loop2, 0)
    wait_copyout(NB - 1, 1)


_zdiff = pl.kernel(
    _zdiff_body,
    out_type=jax.ShapeDtypeStruct((EPAD, L), jnp.float32),
    mesh=_mesh,
    compiler_params=_sc_params,
    scratch_types=[
        pltpu.VMEM((2, 3, SUB), jnp.int32),
        pltpu.VMEM((2, SUB, L), jnp.float32),
        pltpu.VMEM((2, SUB, L), jnp.float32),
        pltpu.SemaphoreType.DMA((2,)),
        pltpu.SemaphoreType.DMA((2,)),
        pltpu.SemaphoreType.DMA((2,)),
        pltpu.SemaphoreType.DMA((2,)),
    ],
)


BM = 1000


def _mm_bias_kernel(x_ref, w_ref, b_ref, o_ref):
    o_ref[...] = (jnp.dot(x_ref[...], w_ref[...],
                          preferred_element_type=jnp.float32) + b_ref[...])


def _mm_bias(x, w, b):
    m, d = x.shape
    h = w.shape[1]
    return pl.pallas_call(
        _mm_bias_kernel,
        grid=(m // BM,),
        in_specs=[
            pl.BlockSpec((BM, d), lambda i: (i, 0)),
            pl.BlockSpec((d, h), lambda i: (0, 0)),
            pl.BlockSpec((1, h), lambda i: (0, 0)),
        ],
        out_specs=pl.BlockSpec((BM, h), lambda i: (i, 0)),
        out_shape=jax.ShapeDtypeStruct((m, h), jnp.float32),
    )(x, w, b.reshape(1, h))


def _relu_mm_kernel(p_ref, w_ref, b_ref, o_ref):
    h = jax.nn.relu(p_ref[0] + p_ref[1])
    o_ref[...] = (jnp.dot(h, w_ref[...],
                          preferred_element_type=jnp.float32) + b_ref[...])


def _relu_mm(p, w, b):
    h = w.shape[1]
    return pl.pallas_call(
        _relu_mm_kernel,
        grid=(N // BM,),
        in_specs=[
            pl.BlockSpec((2, BM, H), lambda i: (0, i, 0)),
            pl.BlockSpec((H, h), lambda i: (0, 0)),
            pl.BlockSpec((1, h), lambda i: (0, 0)),
        ],
        out_specs=pl.BlockSpec((BM, h), lambda i: (i, 0)),
        out_shape=jax.ShapeDtypeStruct((N, h), jnp.float32),
    )(p, w, b.reshape(1, h))


def _heads_kernel(p_ref, eps_ref, wmu_ref, bmu_ref, wlv_ref, blv_ref,
                  fw1_ref, fb1_ref, fw2_ref, fb2_ref, fw3_ref, fb3_ref,
                  mu_ref, lv_ref, z_ref, xr_ref):
    h2 = jax.nn.relu(p_ref[0] + p_ref[1])
    mu = jnp.dot(h2, wmu_ref[...], preferred_element_type=jnp.float32) + bmu_ref[...]
    lv = jnp.dot(h2, wlv_ref[...], preferred_element_type=jnp.float32) + blv_ref[...]
    z = mu + jnp.exp(0.5 * lv) * eps_ref[...]
    mu_ref[...] = mu
    lv_ref[...] = lv
    z_ref[...] = z
    hx = jax.nn.relu(jnp.dot(z, fw1_ref[...], preferred_element_type=jnp.float32)
                     + fb1_ref[...])
    hx = jax.nn.relu(jnp.dot(hx, fw2_ref[...], preferred_element_type=jnp.float32)
                     + fb2_ref[...])
    xr_ref[...] = (jnp.dot(hx, fw3_ref[...], preferred_element_type=jnp.float32)
                   + fb3_ref[...])


def _heads(p, eps, wmu, bmu, wlv, blv, fw1, fb1, fw2, fb2, fw3, fb3):
    f1 = fw1.shape[1]
    return pl.pallas_call(
        _heads_kernel,
        grid=(N // BM,),
        in_specs=[
            pl.BlockSpec((2, BM, H), lambda i: (0, i, 0)),
            pl.BlockSpec((BM, L), lambda i: (i, 0)),
            pl.BlockSpec((H, L), lambda i: (0, 0)),
            pl.BlockSpec((1, L), lambda i: (0, 0)),
            pl.BlockSpec((H, L), lambda i: (0, 0)),
            pl.BlockSpec((1, L), lambda i: (0, 0)),
            pl.BlockSpec((L, f1), lambda i: (0, 0)),
            pl.BlockSpec((1, f1), lambda i: (0, 0)),
            pl.BlockSpec((f1, f1), lambda i: (0, 0)),
            pl.BlockSpec((1, f1), lambda i: (0, 0)),
            pl.BlockSpec((f1, D), lambda i: (0, 0)),
            pl.BlockSpec((1, D), lambda i: (0, 0)),
        ],
        out_specs=[
            pl.BlockSpec((BM, L), lambda i: (i, 0)),
            pl.BlockSpec((BM, L), lambda i: (i, 0)),
            pl.BlockSpec((BM, L), lambda i: (i, 0)),
            pl.BlockSpec((BM, D), lambda i: (i, 0)),
        ],
        out_shape=[
            jax.ShapeDtypeStruct((N, L), jnp.float32),
            jax.ShapeDtypeStruct((N, L), jnp.float32),
            jax.ShapeDtypeStruct((N, L), jnp.float32),
            jax.ShapeDtypeStruct((N, D), jnp.float32),
        ],
    )(p, eps, wmu, bmu.reshape(1, L), wlv, blv.reshape(1, L),
      fw1, fb1.reshape(1, f1), fw2, fb2.reshape(1, f1), fw3, fb3.reshape(1, D))


EBM = 4096


def _edge_logits_kernel(zd_ref, s_ref, la_ref, db_ref, o_ref):
    d = zd_ref[...]
    dist2 = jnp.dot(d * d, s_ref[...], preferred_element_type=jnp.float32)
    la = la_ref[0, 0]
    alpha = jnp.maximum(la, 0.0) + jnp.log1p(jnp.exp(-jnp.abs(la))) + 0.0001
    o_ref[...] = db_ref[0, 0] - alpha * dist2


def _edge_logits(zd8, smat, log_alpha, dec_bias):
    g = EPAD // 8
    return pl.pallas_call(
        _edge_logits_kernel,
        grid=(g // EBM,),
        in_specs=[
            pl.BlockSpec((EBM, 128), lambda i: (i, 0)),
            pl.BlockSpec((128, 8), lambda i: (0, 0)),
            pl.BlockSpec((1, 1), lambda i: (0, 0), memory_space=pltpu.SMEM),
            pl.BlockSpec((1, 1), lambda i: (0, 0), memory_space=pltpu.SMEM),
        ],
        out_specs=pl.BlockSpec((EBM, 8), lambda i: (i, 0)),
        out_shape=jax.ShapeDtypeStruct((g, 8), jnp.float32),
    )(zd8, smat, log_alpha.reshape(1, 1), dec_bias.reshape(1, 1))


def kernel(feats, edge_index, edge_weight, eps, w1, b1, w2, b2, wmu, bmu,
           wlv, blv, log_alpha, dec_bias, fw1, fb1, fw2, fb2, fw3, fb3):
    ei = edge_index.astype(jnp.int32)
    npad = EPAD - E
    zpad_i = jnp.zeros((npad,), jnp.int32)
    sp = jnp.concatenate([ei[0], zpad_i]).reshape(NSUBP, SUB)
    dp = jnp.concatenate([ei[1], zpad_i]).reshape(NSUBP, SUB)
    wp = jnp.concatenate([edge_weight.astype(jnp.float32),
                          jnp.zeros((npad,), jnp.float32)])
    wbits = lax.bitcast_convert_type(wp, jnp.int32).reshape(NSUBP, SUB)
    sdw = jnp.stack([sp, dp, wbits], axis=1)

    hw1 = _mm_bias(feats, w1, b1)
    p1 = _seg_sum(hw1, sdw)
    hw2 = _relu_mm(p1, w2, b2)
    p2 = _seg_sum(hw2, sdw)
    mu, logvar, z, x_recon = _heads(
        p2, eps, wmu, bmu, wlv, blv,
        fw1, fb1, fw2, fb2, fw3, fb3)
    zd = _zdiff(z, sdw)
    smat = jnp.kron(jnp.eye(8, dtype=jnp.float32),
                    jnp.ones((16, 1), dtype=jnp.float32))
    logits8 = _edge_logits(zd.reshape(EPAD // 8, 128),
                           smat, log_alpha, dec_bias)
    edge_logits = logits8.reshape(EPAD)[:E]
    return (edge_logits, x_recon, mu, logvar)


# speedup vs baseline: 1.0512x; 1.0512x over previous
import functools

import jax
import jax.numpy as jnp
from jax import lax
from jax.experimental import pallas as pl
from jax.experimental.pallas import tpu as pltpu
from jax.experimental.pallas import tpu_sc as plsc

N = 10000
E = 320000
D = 128
H = 64
L = 16

SUB = 128
NSUBP = 2560
EPAD = NSUBP * SUB
NB = NSUBP // 32
NPAD = 10240
ROWS_PER_TILE = NPAD // 16
HK = H // 16

_mesh = plsc.VectorSubcoreMesh(core_axis_name="c", subcore_axis_name="s")
_sc_params = pltpu.CompilerParams(use_tc_tiling_on_sc=False,
                                  needs_layout_passes=False)


def _seg_sum_body(hw_hbm, sdw_hbm, out_hbm, ebuf, dbuf, rows, acc,
                  esem, gsem, ssem):
    c = lax.axis_index("c")
    s = lax.axis_index("s")
    wid = s * 2 + c

    z16 = jnp.zeros((16,), jnp.float32)

    def zero_body(i, _):
        for k in range(HK):
            rows[0, i, pl.ds(k * 16, 16)] = z16
        return 0

    lax.fori_loop(0, SUB, zero_body, 0)
    for j in range(ROWS_PER_TILE // SUB):
        pltpu.sync_copy(rows.at[0],
                        acc.at[pl.ds(s * ROWS_PER_TILE + j * SUB, SUB)])
    plsc.subcore_barrier()

    def q_of(t):
        return wid + 32 * t

    def issue_edata(t, u):
        pltpu.async_copy(sdw_hbm.at[q_of(t)], ebuf.at[u], esem.at[u])

    def wait_edata(t, u):
        pltpu.make_async_copy(sdw_hbm.at[q_of(t)], ebuf.at[u],
                              esem.at[u]).wait()

    def issue_gather(u):
        pltpu.async_copy(hw_hbm.at[ebuf.at[u, 0]], rows.at[u], gsem.at[u])

    def wait_gather(u):
        pltpu.make_async_copy(hw_hbm.at[ebuf.at[u, 0]], rows.at[u],
                              gsem.at[u]).wait()

    def issue_scatter(u):
        pltpu.async_copy(rows.at[u], acc.at[dbuf.at[u]], ssem.at[u], add=True)

    def wait_scatter(u):
        pltpu.make_async_copy(rows.at[u], acc.at[dbuf.at[u]],
                              ssem.at[u]).wait()

    def scale(u):
        for g in range(SUB // 16):
            w16 = plsc.bitcast(ebuf[u, 2, pl.ds(g * 16, 16)], jnp.float32)
            for i in range(16):
                wb = w16.at[jnp.full((16,), i, jnp.int32)].get(
                    mode="promise_in_bounds")
                e = g * 16 + i
                for k in range(HK):
                    sl = pl.ds(k * 16, 16)
                    rows[u, e, sl] = rows[u, e, sl] * wb

    for u in range(4):
        issue_edata(u, u)
    wait_edata(0, 0)
    issue_gather(0)
    wait_edata(1, 1)
    issue_gather(1)

    def step(t4, u):
        t = t4 * 4 + u
        wait_gather(u)
        for g in range(SUB // 16):
            sl = pl.ds(g * 16, 16)
            dbuf[u, sl] = ebuf[u, 1, sl]
        scale(u)
        issue_scatter(u)

        @pl.when(t4 < (NB // 4) - 1)
        def _():
            issue_edata(t + 4, u)

        un2 = (u + 2) % 4

        def tail_ops(with_scatter_wait):
            if with_scatter_wait:
                wait_scatter(un2)
            wait_edata(t + 2, un2)
            issue_gather(un2)

        if u >= 2:
            @pl.when(t4 < (NB // 4) - 1)
            def _():
                tail_ops(True)
        else:
            @pl.when(t4 >= 1)
            def _():
                tail_ops(True)

            @pl.when(t4 == 0)
            def _():
                tail_ops(False)

    def loop_body(t4, _):
        for u in range(4):
            step(t4, u)
        return 0

    lax.fori_loop(0, NB // 4, loop_body, 0)

    for u in range(4):
        wait_scatter(u)
    plsc.subcore_barrier()
    pltpu.sync_copy(acc.at[pl.ds(s * ROWS_PER_TILE, ROWS_PER_TILE)],
                    out_hbm.at[c, pl.ds(s * ROWS_PER_TILE, ROWS_PER_TILE)])


_seg_sum = pl.kernel(
    _seg_sum_body,
    out_type=jax.ShapeDtypeStruct((2, NPAD, H), jnp.float32),
    mesh=_mesh,
    compiler_params=_sc_params,
    scratch_types=[
        pltpu.VMEM((4, 3, SUB), jnp.int32),
        pltpu.VMEM((4, SUB), jnp.int32),
        pltpu.VMEM((4, SUB, H), jnp.float32),
        pltpu.VMEM_SHARED((NPAD, H), jnp.float32),
        pltpu.SemaphoreType.DMA((4,)),
        pltpu.SemaphoreType.DMA((4,)),
        pltpu.SemaphoreType.DMA((4,)),
    ],
)


def _zdiff_body(z_hbm, sdw_hbm, zd_hbm, ebuf, zib, zjb,
                esem, gisem, gjsem, csem):
    c = lax.axis_index("c")
    s = lax.axis_index("s")
    wid = s * 2 + c

    def q_of(t):
        return wid + 32 * t

    def issue_edata(t, u):
        pltpu.async_copy(sdw_hbm.at[q_of(t)], ebuf.at[u], esem.at[u])

    def wait_edata(t, u):
        pltpu.make_async_copy(sdw_hbm.at[q_of(t)], ebuf.at[u],
                              esem.at[u]).wait()

    def issue_gathers(u):
        pltpu.async_copy(z_hbm.at[ebuf.at[u, 0]], zib.at[u], gisem.at[u])
        pltpu.async_copy(z_hbm.at[ebuf.at[u, 1]], zjb.at[u], gjsem.at[u])

    def wait_gathers(u):
        pltpu.make_async_copy(z_hbm.at[ebuf.at[u, 0]], zib.at[u],
                              gisem.at[u]).wait()
        pltpu.make_async_copy(z_hbm.at[ebuf.at[u, 1]], zjb.at[u],
                              gjsem.at[u]).wait()

    def issue_copyout(t, u):
        pltpu.async_copy(zib.at[u], zd_hbm.at[pl.ds(q_of(t) * SUB, SUB)],
                         csem.at[u])

    def wait_copyout(t, u):
        pltpu.make_async_copy(zib.at[u], zd_hbm.at[pl.ds(q_of(t) * SUB, SUB)],
                              csem.at[u]).wait()

    for u in range(4):
        issue_edata(u, u)
    wait_edata(0, 0)
    issue_gathers(0)
    wait_edata(1, 1)
    issue_gathers(1)

    def step(t4, u):
        t = t4 * 4 + u
        wait_gathers(u)
        for e in range(SUB):
            zib[u, e, :] = zib[u, e, :] - zjb[u, e, :]
        issue_copyout(t, u)

        @pl.when(t4 < (NB // 4) - 1)
        def _():
            issue_edata(t + 4, u)

        un2 = (u + 2) % 4

        def tail_ops(with_copyout_wait):
            if with_copyout_wait:
                wait_copyout(t - 2, un2)
            wait_edata(t + 2, un2)
            issue_gathers(un2)

        if u >= 2:
            @pl.when(t4 < (NB // 4) - 1)
            def _():
                tail_ops(True)
        else:
            @pl.when(t4 >= 1)
            def _():
                tail_ops(True)

            @pl.when(t4 == 0)
            def _():
                tail_ops(False)

    def loop_body(t4, _):
        for u in range(4):
            step(t4, u)
        return 0

    lax.fori_loop(0, NB // 4, loop_body, 0)

    for u in range(4):
        wait_copyout(NB - 4 + u, u)


_zdiff = pl.kernel(
    _zdiff_body,
    out_type=jax.ShapeDtypeStruct((EPAD, L), jnp.float32),
    mesh=_mesh,
    compiler_params=_sc_params,
    scratch_types=[
        pltpu.VMEM((4, 3, SUB), jnp.int32),
        pltpu.VMEM((4, SUB, L), jnp.float32),
        pltpu.VMEM((4, SUB, L), jnp.float32),
        pltpu.SemaphoreType.DMA((4,)),
        pltpu.SemaphoreType.DMA((4,)),
        pltpu.SemaphoreType.DMA((4,)),
        pltpu.SemaphoreType.DMA((4,)),
    ],
)


BM = 1000


def _mm_bias_kernel(x_ref, w_ref, b_ref, o_ref):
    o_ref[...] = (jnp.dot(x_ref[...], w_ref[...],
                          preferred_element_type=jnp.float32) + b_ref[...])


def _mm_bias(x, w, b):
    m, d = x.shape
    h = w.shape[1]
    return pl.pallas_call(
        _mm_bias_kernel,
        grid=(m // BM,),
        in_specs=[
            pl.BlockSpec((BM, d), lambda i: (i, 0)),
            pl.BlockSpec((d, h), lambda i: (0, 0)),
            pl.BlockSpec((1, h), lambda i: (0, 0)),
        ],
        out_specs=pl.BlockSpec((BM, h), lambda i: (i, 0)),
        out_shape=jax.ShapeDtypeStruct((m, h), jnp.float32),
    )(x, w, b.reshape(1, h))


def _relu_mm_kernel(p_ref, w_ref, b_ref, o_ref):
    h = jax.nn.relu(p_ref[0] + p_ref[1])
    o_ref[...] = (jnp.dot(h, w_ref[...],
                          preferred_element_type=jnp.float32) + b_ref[...])


def _relu_mm(p, w, b):
    h = w.shape[1]
    return pl.pallas_call(
        _relu_mm_kernel,
        grid=(N // BM,),
        in_specs=[
            pl.BlockSpec((2, BM, H), lambda i: (0, i, 0)),
            pl.BlockSpec((H, h), lambda i: (0, 0)),
            pl.BlockSpec((1, h), lambda i: (0, 0)),
        ],
        out_specs=pl.BlockSpec((BM, h), lambda i: (i, 0)),
        out_shape=jax.ShapeDtypeStruct((N, h), jnp.float32),
    )(p, w, b.reshape(1, h))


def _heads_kernel(p_ref, eps_ref, wmu_ref, bmu_ref, wlv_ref, blv_ref,
                  fw1_ref, fb1_ref, fw2_ref, fb2_ref, fw3_ref, fb3_ref,
                  mu_ref, lv_ref, z_ref, xr_ref):
    h2 = jax.nn.relu(p_ref[0] + p_ref[1])
    mu = jnp.dot(h2, wmu_ref[...], preferred_element_type=jnp.float32) + bmu_ref[...]
    lv = jnp.dot(h2, wlv_ref[...], preferred_element_type=jnp.float32) + blv_ref[...]
    z = mu + jnp.exp(0.5 * lv) * eps_ref[...]
    mu_ref[...] = mu
    lv_ref[...] = lv
    z_ref[...] = z
    hx = jax.nn.relu(jnp.dot(z, fw1_ref[...], preferred_element_type=jnp.float32)
                     + fb1_ref[...])
    hx = jax.nn.relu(jnp.dot(hx, fw2_ref[...], preferred_element_type=jnp.float32)
                     + fb2_ref[...])
    xr_ref[...] = (jnp.dot(hx, fw3_ref[...], preferred_element_type=jnp.float32)
                   + fb3_ref[...])


def _heads(p, eps, wmu, bmu, wlv, blv, fw1, fb1, fw2, fb2, fw3, fb3):
    f1 = fw1.shape[1]
    return pl.pallas_call(
        _heads_kernel,
        grid=(N // BM,),
        in_specs=[
            pl.BlockSpec((2, BM, H), lambda i: (0, i, 0)),
            pl.BlockSpec((BM, L), lambda i: (i, 0)),
            pl.BlockSpec((H, L), lambda i: (0, 0)),
            pl.BlockSpec((1, L), lambda i: (0, 0)),
            pl.BlockSpec((H, L), lambda i: (0, 0)),
            pl.BlockSpec((1, L), lambda i: (0, 0)),
            pl.BlockSpec((L, f1), lambda i: (0, 0)),
            pl.BlockSpec((1, f1), lambda i: (0, 0)),
            pl.BlockSpec((f1, f1), lambda i: (0, 0)),
            pl.BlockSpec((1, f1), lambda i: (0, 0)),
            pl.BlockSpec((f1, D), lambda i: (0, 0)),
            pl.BlockSpec((1, D), lambda i: (0, 0)),
        ],
        out_specs=[
            pl.BlockSpec((BM, L), lambda i: (i, 0)),
            pl.BlockSpec((BM, L), lambda i: (i, 0)),
            pl.BlockSpec((BM, L), lambda i: (i, 0)),
            pl.BlockSpec((BM, D), lambda i: (i, 0)),
        ],
        out_shape=[
            jax.ShapeDtypeStruct((N, L), jnp.float32),
            jax.ShapeDtypeStruct((N, L), jnp.float32),
            jax.ShapeDtypeStruct((N, L), jnp.float32),
            jax.ShapeDtypeStruct((N, D), jnp.float32),
        ],
    )(p, eps, wmu, bmu.reshape(1, L), wlv, blv.reshape(1, L),
      fw1, fb1.reshape(1, f1), fw2, fb2.reshape(1, f1), fw3, fb3.reshape(1, D))


EBM = 4096


def _edge_logits_kernel(zd_ref, s_ref, la_ref, db_ref, o_ref):
    d = zd_ref[...]
    dist2 = jnp.dot(d * d, s_ref[...], preferred_element_type=jnp.float32)
    la = la_ref[0, 0]
    alpha = jnp.maximum(la, 0.0) + jnp.log1p(jnp.exp(-jnp.abs(la))) + 0.0001
    o_ref[...] = db_ref[0, 0] - alpha * dist2


def _edge_logits(zd8, smat, log_alpha, dec_bias):
    g = EPAD // 8
    return pl.pallas_call(
        _edge_logits_kernel,
        grid=(g // EBM,),
        in_specs=[
            pl.BlockSpec((EBM, 128), lambda i: (i, 0)),
            pl.BlockSpec((128, 8), lambda i: (0, 0)),
            pl.BlockSpec((1, 1), lambda i: (0, 0), memory_space=pltpu.SMEM),
            pl.BlockSpec((1, 1), lambda i: (0, 0), memory_space=pltpu.SMEM),
        ],
        out_specs=pl.BlockSpec((EBM, 8), lambda i: (i, 0)),
        out_shape=jax.ShapeDtypeStruct((g, 8), jnp.float32),
    )(zd8, smat, log_alpha.reshape(1, 1), dec_bias.reshape(1, 1))


def kernel(feats, edge_index, edge_weight, eps, w1, b1, w2, b2, wmu, bmu,
           wlv, blv, log_alpha, dec_bias, fw1, fb1, fw2, fb2, fw3, fb3):
    ei = edge_index.astype(jnp.int32)
    npad = EPAD - E
    zpad_i = jnp.zeros((npad,), jnp.int32)
    sp = jnp.concatenate([ei[0], zpad_i]).reshape(NSUBP, SUB)
    dp = jnp.concatenate([ei[1], zpad_i]).reshape(NSUBP, SUB)
    wp = jnp.concatenate([edge_weight.astype(jnp.float32),
                          jnp.zeros((npad,), jnp.float32)])
    wbits = lax.bitcast_convert_type(wp, jnp.int32).reshape(NSUBP, SUB)
    sdw = jnp.stack([sp, dp, wbits], axis=1)

    hw1 = _mm_bias(feats, w1, b1)
    p1 = _seg_sum(hw1, sdw)
    hw2 = _relu_mm(p1, w2, b2)
    p2 = _seg_sum(hw2, sdw)
    mu, logvar, z, x_recon = _heads(
        p2, eps, wmu, bmu, wlv, blv,
        fw1, fb1, fw2, fb2, fw3, fb3)
    zd = _zdiff(z, sdw)
    smat = jnp.kron(jnp.eye(8, dtype=jnp.float32),
                    jnp.ones((16, 1), dtype=jnp.float32))
    logits8 = _edge_logits(zd.reshape(EPAD // 8, 128),
                           smat, log_alpha, dec_bias)
    edge_logits = logits8.reshape(EPAD)[:E]
    return (edge_logits, x_recon, mu, logvar)
